# baseline (speedup 1.0000x reference)
import functools

import jax
import jax.numpy as jnp
from jax import lax
from jax.experimental import pallas as pl
from jax.experimental.pallas import tpu as pltpu
from jax.experimental.pallas import tpu_sc as plsc

N = 10000
E = 320000
F_IN = 128
H = 64
C = 10
G = 64

NC = 2
NS = 16
NW = NC * NS
NPAD = 10240

EC = 80
CPW = E // NW // EC
NB = 5
EPT = E // NW

BR = 2000
NACC = 10240
RPT = NACC // NS

_mesh = plsc.VectorSubcoreMesh(core_axis_name="c", subcore_axis_name="s")
_sc_params = pltpu.CompilerParams(needs_layout_passes=False,
                                  use_tc_tiling_on_sc=False)


@functools.partial(
    pl.kernel,
    mesh=_mesh,
    out_type=jax.ShapeDtypeStruct((NW, NPAD), jnp.float32),
    scratch_types=[
        pltpu.VMEM((NPAD,), jnp.float32),
        pltpu.VMEM((EPT,), jnp.int32),
        pltpu.SemaphoreType.DMA,
    ],
    compiler_params=_sc_params,
)
def _sc_hist(dst_hbm, out_hbm, hist_v, chunk_v, sem):
    c = lax.axis_index("c")
    s = lax.axis_index("s")
    wid = s * NC + c
    cp = pltpu.async_copy(dst_hbm.at[pl.ds(wid * EPT, EPT)], chunk_v, sem)
    zero16 = jnp.zeros((16,), jnp.float32)

    def zbody(j, carry):
        hist_v[pl.ds(j * 16, 16)] = zero16
        return carry

    lax.fori_loop(0, NPAD // 16, zbody, 0)
    cp.wait()
    ones16 = jnp.ones((16,), jnp.float32)

    def inner(j, c2):
        idx = chunk_v[pl.ds(j * 16, 16)]
        plsc.addupdate_scatter(hist_v, [idx], ones16)
        return c2

    lax.fori_loop(0, EPT // 16, inner, 0)
    pltpu.sync_copy(hist_v, out_hbm.at[wid])


@functools.partial(
    pl.kernel,
    mesh=_mesh,
    out_type=jax.ShapeDtypeStruct((NC, NACC, H), jnp.float32),
    scratch_types=[
        pltpu.VMEM((CPW, EC), jnp.int32),
        pltpu.VMEM((CPW, EC), jnp.int32),
        [pltpu.VMEM((EC, H), jnp.float32)] * NB,
        pltpu.VMEM_SHARED((NACC, H), jnp.float32),
        pltpu.VMEM_SHARED((N, H), jnp.float32),
        pltpu.SemaphoreType.DMA,
        [pltpu.SemaphoreType.DMA] * NB,
    ],
    compiler_params=_sc_params,
)
def _sc_agg(hp_hbm, src_hbm, dst_hbm, zeros_hbm, out_hbm,
            srcb, dstb, rows, accum_sh, hp_sh, isem, gsems):
    c = lax.axis_index("c")
    s = lax.axis_index("s")
    wid = s * NC + c

    icp1 = pltpu.async_copy(src_hbm.at[pl.ds(wid * CPW, CPW)], srcb, isem)
    icp2 = pltpu.async_copy(dst_hbm.at[pl.ds(wid * CPW, CPW)], dstb, isem)

    pltpu.sync_copy(hp_hbm.at[pl.ds(s * (N // NS), N // NS)],
                    hp_sh.at[pl.ds(s * (N // NS), N // NS)])
    pltpu.sync_copy(zeros_hbm.at[pl.ds(s * RPT, RPT)],
                    accum_sh.at[pl.ds(s * RPT, RPT)])
    icp1.wait()
    icp2.wait()
    plsc.subcore_barrier()

    for b in range(NB):
        pltpu.async_copy(hp_sh.at[srcb.at[b]], rows[b], gsems[b])

    def body(i, carry):
        j0 = i * NB
        for b in range(NB):
            j = j0 + b
            pltpu.make_async_copy(hp_sh.at[srcb.at[j]], rows[b],
                                  gsems[b]).wait()
            pltpu.sync_copy(rows[b], accum_sh.at[dstb.at[j]], add=True)

            @pl.when(j + NB < CPW)
            def _():
                pltpu.async_copy(hp_sh.at[srcb.at[j + NB]], rows[b],
                                 gsems[b])
        return carry

    lax.fori_loop(0, CPW // NB, body, 0)
    plsc.subcore_barrier()
    pltpu.sync_copy(accum_sh.at[pl.ds(s * RPT, RPT)],
                    out_hbm.at[c, pl.ds(s * RPT, RPT)])


def _tc_first_body(x_ref, w_ref, degp_ref, hp_ref, dinv_ref):
    deg = jnp.sum(degp_ref[...], axis=1) + 1.0
    dinv = lax.rsqrt(deg)[:, None]
    h = jnp.dot(x_ref[...], w_ref[...], preferred_element_type=jnp.float32)
    hp_ref[...] = h * dinv
    dinv_ref[...] = dinv


def _tc_mid_body(aggp_ref, hp_ref, dinv_ref, b_ref, w_ref, out_ref):
    dinv = dinv_ref[...]
    agg = aggp_ref[0] + aggp_ref[1] + hp_ref[...]
    z = jnp.maximum(agg * dinv + b_ref[...], 0.0)
    out_ref[...] = jnp.dot(
        z, w_ref[...], preferred_element_type=jnp.float32) * dinv


def _tc_pool_body(aggp_ref, hp_ref, dinv_ref, b_ref, batch_ref,
                  linw_ref, linb_ref, out_ref, acc_ref):
    i = pl.program_id(0)
    z = (aggp_ref[0] + aggp_ref[1] + hp_ref[...]) * dinv_ref[...] + b_ref[...]
    bb = batch_ref[...]
    gi = lax.broadcasted_iota(jnp.int32, (BR, G), 1)
    onehot = (gi == bb).astype(jnp.float32)
    zc = jnp.concatenate([z, jnp.ones((BR, 1), jnp.float32)], axis=1)
    part = lax.dot_general(onehot, zc, (((0,), (0,)), ((), ())),
                           preferred_element_type=jnp.float32)

    @pl.when(i == 0)
    def _():
        acc_ref[...] = part

    @pl.when(i > 0)
    def _():
        acc_ref[...] = acc_ref[...] + part

    @pl.when(i == pl.num_programs(0) - 1)
    def _():
        sums = acc_ref[:, :H]
        cnt = acc_ref[:, H:]
        pooled = sums / jnp.maximum(cnt, 1.0)
        out_ref[...] = jnp.dot(
            pooled, linw_ref[...],
            preferred_element_type=jnp.float32) + linb_ref[...]


def _tc_first(x, w1, degp):
    return pl.pallas_call(
        _tc_first_body,
        grid=(N // BR,),
        in_specs=[
            pl.BlockSpec((BR, F_IN), lambda i: (i, 0)),
            pl.BlockSpec((F_IN, H), lambda i: (0, 0)),
            pl.BlockSpec((BR, NW), lambda i: (i, 0)),
        ],
        out_specs=[
            pl.BlockSpec((BR, H), lambda i: (i, 0)),
            pl.BlockSpec((BR, 1), lambda i: (i, 0)),
        ],
        out_shape=[
            jax.ShapeDtypeStruct((N, H), jnp.float32),
            jax.ShapeDtypeStruct((N, 1), jnp.float32),
        ],
    )(x, w1, degp)


def _tc_mid(aggp, hp, dinv, b, w):
    return pl.pallas_call(
        _tc_mid_body,
        grid=(N // BR,),
        in_specs=[
            pl.BlockSpec((NC, BR, H), lambda i: (0, i, 0)),
            pl.BlockSpec((BR, H), lambda i: (i, 0)),
            pl.BlockSpec((BR, 1), lambda i: (i, 0)),
            pl.BlockSpec((1, H), lambda i: (0, 0)),
            pl.BlockSpec((H, H), lambda i: (0, 0)),
        ],
        out_specs=pl.BlockSpec((BR, H), lambda i: (i, 0)),
        out_shape=jax.ShapeDtypeStruct((N, H), jnp.float32),
    )(aggp, hp, dinv, b, w)


def _tc_pool(aggp, hp, dinv, b, batch2, linw, linb):
    return pl.pallas_call(
        _tc_pool_body,
        grid=(N // BR,),
        in_specs=[
            pl.BlockSpec((NC, BR, H), lambda i: (0, i, 0)),
            pl.BlockSpec((BR, H), lambda i: (i, 0)),
            pl.BlockSpec((BR, 1), lambda i: (i, 0)),
            pl.BlockSpec((1, H), lambda i: (0, 0)),
            pl.BlockSpec((BR, 1), lambda i: (i, 0)),
            pl.BlockSpec((H, C), lambda i: (0, 0)),
            pl.BlockSpec((1, C), lambda i: (0, 0)),
        ],
        out_specs=pl.BlockSpec((G, C), lambda i: (0, 0)),
        out_shape=jax.ShapeDtypeStruct((G, C), jnp.float32),
        scratch_shapes=[pltpu.VMEM((G, H + 1), jnp.float32)],
    )(aggp, hp, dinv, b, batch2, linw, linb)


def kernel(x, edge_index, batch, W1, b1, W2, b2, W3, b3, lin_W, lin_b):
    src2 = edge_index[0].reshape(E // EC, EC)
    dst = edge_index[1]
    dst2 = dst.reshape(E // EC, EC)
    degp = _sc_hist(dst).T[:N]
    hp1, dinv = _tc_first(x, W1, degp)
    zeros = jnp.zeros((NACC, H), jnp.float32)
    agg1 = _sc_agg(hp1, src2, dst2, zeros)
    hp2 = _tc_mid(agg1, hp1, dinv, b1.reshape(1, H), W2)
    agg2 = _sc_agg(hp2, src2, dst2, zeros)
    hp3 = _tc_mid(agg2, hp2, dinv, b2.reshape(1, H), W3)
    agg3 = _sc_agg(hp3, src2, dst2, zeros)
    return _tc_pool(agg3, hp3, dinv, b3.reshape(1, H), batch.reshape(N, 1),
                    lin_W, lin_b.reshape(1, C))


# speedup vs baseline: 1.5657x; 1.5657x over previous
import functools

import jax
import jax.numpy as jnp
from jax import lax
from jax.experimental import pallas as pl
from jax.experimental.pallas import tpu as pltpu
from jax.experimental.pallas import tpu_sc as plsc

N = 10000
E = 320000
F_IN = 128
H = 64
C = 10
G = 64

NC = 2
NS = 16
NW = NC * NS
NPAD = 10240

EC = 80
CPW = E // NW // EC
NB = 5
EPT = E // NW

BR = 2000
NACC = 10240
RPT = NACC // NS
_BW = BR * H // 128
_VR = N * H // 128

_mesh = plsc.VectorSubcoreMesh(core_axis_name="c", subcore_axis_name="s")
_sc_params = pltpu.CompilerParams(needs_layout_passes=False,
                                  use_tc_tiling_on_sc=False)


@functools.partial(
    pl.kernel,
    mesh=_mesh,
    out_type=jax.ShapeDtypeStruct((NW, NPAD), jnp.float32),
    scratch_types=[
        pltpu.VMEM((NPAD,), jnp.float32),
        pltpu.VMEM((EPT,), jnp.int32),
        pltpu.SemaphoreType.DMA,
    ],
    compiler_params=_sc_params,
)
def _sc_hist(dst_hbm, out_hbm, hist_v, chunk_v, sem):
    c = lax.axis_index("c")
    s = lax.axis_index("s")
    wid = s * NC + c
    cp = pltpu.async_copy(dst_hbm.at[pl.ds(wid * EPT, EPT)], chunk_v, sem)
    zero16 = jnp.zeros((16,), jnp.float32)

    def zbody(j, carry):
        hist_v[pl.ds(j * 16, 16)] = zero16
        return carry

    lax.fori_loop(0, NPAD // 16, zbody, 0)
    cp.wait()
    ones16 = jnp.ones((16,), jnp.float32)

    def inner(j, c2):
        idx = chunk_v[pl.ds(j * 16, 16)]
        plsc.addupdate_scatter(hist_v, [idx], ones16)
        return c2

    lax.fori_loop(0, EPT // 16, inner, 0)
    pltpu.sync_copy(hist_v, out_hbm.at[wid])


@functools.partial(
    pl.kernel,
    mesh=_mesh,
    out_type=jax.ShapeDtypeStruct((NC, NACC, H), jnp.float32),
    scratch_types=[
        pltpu.VMEM((CPW, EC), jnp.int32),
        pltpu.VMEM((CPW, EC), jnp.int32),
        [pltpu.VMEM((EC, H), jnp.float32)] * NB,
        pltpu.VMEM_SHARED((NACC, H), jnp.float32),
        pltpu.SemaphoreType.DMA,
        [pltpu.SemaphoreType.DMA] * NB,
    ],
    compiler_params=_sc_params,
)
def _sc_agg(hp_hbm, src_hbm, dst_hbm, out_hbm,
            srcb, dstb, rows, accum_sh, isem, gsems):
    c = lax.axis_index("c")
    s = lax.axis_index("s")
    wid = s * NC + c

    icp1 = pltpu.async_copy(src_hbm.at[pl.ds(wid * CPW, CPW)], srcb, isem)
    icp2 = pltpu.async_copy(dst_hbm.at[pl.ds(wid * CPW, CPW)], dstb, isem)

    zero16 = jnp.zeros((16,), jnp.float32)

    def zbody(j, carry):
        rows[0][j // (H // 16), pl.ds((j % (H // 16)) * 16, 16)] = zero16
        return carry

    lax.fori_loop(0, EC * H // 16, zbody, 0)

    @pl.when((c == 0) & (s < NS - 1))
    def _():
        pltpu.sync_copy(hp_hbm.at[pl.ds(s * RPT, RPT)],
                        accum_sh.at[pl.ds(s * RPT, RPT)])

    @pl.when((c == 0) & (s == NS - 1))
    def _():
        pltpu.sync_copy(hp_hbm.at[pl.ds((NS - 1) * RPT, N - (NS - 1) * RPT)],
                        accum_sh.at[pl.ds((NS - 1) * RPT, N - (NS - 1) * RPT)])
        for q in range((NACC - N) // EC):
            pltpu.sync_copy(rows[0], accum_sh.at[pl.ds(N + q * EC, EC)])

    @pl.when(c == 1)
    def _():
        for q in range(RPT // EC):
            pltpu.sync_copy(rows[0], accum_sh.at[pl.ds(s * RPT + q * EC, EC)])

    icp1.wait()
    icp2.wait()
    plsc.subcore_barrier()

    for b in range(NB):
        pltpu.async_copy(hp_hbm.at[srcb.at[b]], rows[b], gsems[b])

    def body(i, carry):
        j0 = i * NB
        for b in range(NB):
            j = j0 + b
            pltpu.make_async_copy(hp_hbm.at[srcb.at[j]], rows[b],
                                  gsems[b]).wait()
            pltpu.sync_copy(rows[b], accum_sh.at[dstb.at[j]], add=True)

            @pl.when(j + NB < CPW)
            def _():
                pltpu.async_copy(hp_hbm.at[srcb.at[j + NB]], rows[b],
                                 gsems[b])
        return carry

    lax.fori_loop(0, CPW // NB, body, 0)
    plsc.subcore_barrier()
    pltpu.sync_copy(accum_sh.at[pl.ds(s * RPT, RPT)],
                    out_hbm.at[c, pl.ds(s * RPT, RPT)])


def _pair_expand(dvp):
    r2 = lax.broadcasted_iota(jnp.int32, (2, 128), 0)
    ll = lax.broadcasted_iota(jnp.int32, (2, 128), 1)
    sel = (ll // 64 == r2).astype(jnp.float32)
    return lax.dot_general(dvp, sel, (((1,), (0,)), ((), ())),
                           preferred_element_type=jnp.float32)


def _tc_dinv_body(degp_ref, dinv_ref):
    v = jnp.reshape(degp_ref[...], (NW, NPAD // 128, 128))
    deg = jnp.sum(v, axis=0) + 1.0
    dinv_ref[...] = lax.rsqrt(deg)


def _tc_first_body(xv_ref, w_ref, dvp_ref, hp_ref):
    dinvw = _pair_expand(dvp_ref[...])
    h = jnp.dot(xv_ref[...], w_ref[...], preferred_element_type=jnp.float32)
    hp_ref[...] = h * dinvw


def _tc_mid_body(aggp_ref, dvp_ref, b_ref, w_ref, out_ref):
    dinvw = _pair_expand(dvp_ref[...])
    a = aggp_ref[...]
    agg = a[0] + a[1]
    z = jnp.maximum(agg * dinvw + b_ref[...], 0.0)
    out_ref[...] = jnp.dot(
        z, w_ref[...], preferred_element_type=jnp.float32) * dinvw


def _tc_pool_body(aggp_ref, dvp_ref, b_ref, batch_ref,
                  linw_ref, linb_ref, out_ref, acc_ref):
    i = pl.program_id(0)
    dinvw = _pair_expand(dvp_ref[...])
    a = aggp_ref[...]
    z = (a[0] + a[1]) * dinvw + b_ref[...]
    bb = batch_ref[...]
    gi = lax.broadcasted_iota(jnp.int32, (_BW, G), 1)
    oh_e = (gi == bb[:, :1]).astype(jnp.float32)
    oh_o = (gi == bb[:, 1:]).astype(jnp.float32)
    ones = jnp.ones((_BW, 1), jnp.float32)
    zc_e = jnp.concatenate([z[:, :H], ones], axis=1)
    zc_o = jnp.concatenate([z[:, H:], ones], axis=1)
    part = (lax.dot_general(oh_e, zc_e, (((0,), (0,)), ((), ())),
                            preferred_element_type=jnp.float32)
            + lax.dot_general(oh_o, zc_o, (((0,), (0,)), ((), ())),
                              preferred_element_type=jnp.float32))

    @pl.when(i == 0)
    def _():
        acc_ref[...] = part

    @pl.when(i > 0)
    def _():
        acc_ref[...] = acc_ref[...] + part

    @pl.when(i == pl.num_programs(0) - 1)
    def _():
        sums = acc_ref[:, :H]
        cnt = acc_ref[:, H:]
        pooled = sums / jnp.maximum(cnt, 1.0)
        out_ref[...] = jnp.dot(
            pooled, linw_ref[...],
            preferred_element_type=jnp.float32) + linb_ref[...]


def _tc_dinv(degpv):
    return pl.pallas_call(
        _tc_dinv_body,
        grid=(1,),
        in_specs=[pl.BlockSpec((NW * NPAD // 128, 128), lambda i: (0, 0))],
        out_specs=pl.BlockSpec((NPAD // 128, 128), lambda i: (0, 0)),
        out_shape=jax.ShapeDtypeStruct((NPAD // 128, 128), jnp.float32),
    )(degpv)


def _tc_first(xv, wblk, dvp):
    return pl.pallas_call(
        _tc_first_body,
        grid=(N // BR,),
        in_specs=[
            pl.BlockSpec((_BW, 2 * F_IN), lambda i: (i, 0)),
            pl.BlockSpec((2 * F_IN, 128), lambda i: (0, 0)),
            pl.BlockSpec((_BW, 2), lambda i: (i, 0)),
        ],
        out_specs=pl.BlockSpec((_BW, 128), lambda i: (i, 0)),
        out_shape=jax.ShapeDtypeStruct((_VR, 128), jnp.float32),
    )(xv, wblk, dvp)


def _tc_mid(aggv, dvp, bb, wblk):
    return pl.pallas_call(
        _tc_mid_body,
        grid=(N // BR,),
        in_specs=[
            pl.BlockSpec((NC, _BW, 128), lambda i: (0, i, 0)),
            pl.BlockSpec((_BW, 2), lambda i: (i, 0)),
            pl.BlockSpec((1, 128), lambda i: (0, 0)),
            pl.BlockSpec((128, 128), lambda i: (0, 0)),
        ],
        out_specs=pl.BlockSpec((_BW, 128), lambda i: (i, 0)),
        out_shape=jax.ShapeDtypeStruct((_VR, 128), jnp.float32),
    )(aggv, dvp, bb, wblk)


def _tc_pool(aggv, dvp, bb, batch2, linw, linb):
    return pl.pallas_call(
        _tc_pool_body,
        grid=(N // BR,),
        in_specs=[
            pl.BlockSpec((NC, _BW, 128), lambda i: (0, i, 0)),
            pl.BlockSpec((_BW, 2), lambda i: (i, 0)),
            pl.BlockSpec((1, 128), lambda i: (0, 0)),
            pl.BlockSpec((_BW, 2), lambda i: (i, 0)),
            pl.BlockSpec((H, C), lambda i: (0, 0)),
            pl.BlockSpec((1, C), lambda i: (0, 0)),
        ],
        out_specs=pl.BlockSpec((G, C), lambda i: (0, 0)),
        out_shape=jax.ShapeDtypeStruct((G, C), jnp.float32),
        scratch_shapes=[pltpu.VMEM((G, H + 1), jnp.float32)],
    )(aggv, dvp, bb, batch2, linw, linb)


def _blockdiag(w):
    k, m = w.shape
    out = jnp.zeros((2 * k, 2 * m), jnp.float32)
    return out.at[:k, :m].set(w).at[k:, m:].set(w)


def kernel(x, edge_index, batch, W1, b1, W2, b2, W3, b3, lin_W, lin_b):
    src2 = edge_index[0].reshape(E // EC, EC)
    dst = edge_index[1]
    dst2 = dst.reshape(E // EC, EC)
    degpv = _sc_hist(dst).reshape(NW * NPAD // 128, 128)
    dvp = _tc_dinv(degpv).reshape(NPAD)[:N].reshape(N // 2, 2)
    xv = x.reshape(N // 2, 2 * F_IN)
    bb1 = jnp.concatenate([b1, b1]).reshape(1, 128)
    bb2 = jnp.concatenate([b2, b2]).reshape(1, 128)
    bb3 = jnp.concatenate([b3, b3]).reshape(1, 128)
    hpw1 = _tc_first(xv, _blockdiag(W1), dvp)
    agg1 = _sc_agg(hpw1.reshape(N, H), src2, dst2)
    hpw2 = _tc_mid(agg1.reshape(NC, NACC * H // 128, 128), dvp,
                   bb1, _blockdiag(W2))
    agg2 = _sc_agg(hpw2.reshape(N, H), src2, dst2)
    hpw3 = _tc_mid(agg2.reshape(NC, NACC * H // 128, 128), dvp,
                   bb2, _blockdiag(W3))
    agg3 = _sc_agg(hpw3.reshape(N, H), src2, dst2)
    return _tc_pool(agg3.reshape(NC, NACC * H // 128, 128), dvp,
                    bb3, batch.reshape(N // 2, 2),
                    lin_W, lin_b.reshape(1, C))


# speedup vs baseline: 1.5944x; 1.0183x over previous
import functools

import jax
import jax.numpy as jnp
from jax import lax
from jax.experimental import pallas as pl
from jax.experimental.pallas import tpu as pltpu
from jax.experimental.pallas import tpu_sc as plsc

N = 10000
E = 320000
F_IN = 128
H = 64
C = 10
G = 64

NC = 2
NS = 16
NW = NC * NS
NPAD = 10240

EC = 80
CPW = E // NW // EC
NB = 5
EPT = E // NW

BR = 2000
NACC = 10240
RPT = NACC // NS
_BW = BR * H // 128
_VR = N * H // 128

_mesh = plsc.VectorSubcoreMesh(core_axis_name="c", subcore_axis_name="s")
_sc_params = pltpu.CompilerParams(needs_layout_passes=False,
                                  use_tc_tiling_on_sc=False)


@functools.partial(
    pl.kernel,
    mesh=_mesh,
    out_type=jax.ShapeDtypeStruct((NW, NPAD), jnp.float32),
    scratch_types=[
        pltpu.VMEM((NPAD,), jnp.float32),
        pltpu.VMEM((EPT,), jnp.int32),
        pltpu.SemaphoreType.DMA,
    ],
    compiler_params=_sc_params,
)
def _sc_hist(dst_hbm, out_hbm, hist_v, chunk_v, sem):
    c = lax.axis_index("c")
    s = lax.axis_index("s")
    wid = s * NC + c
    cp = pltpu.async_copy(dst_hbm.at[pl.ds(wid * EPT, EPT)], chunk_v, sem)
    zero16 = jnp.zeros((16,), jnp.float32)

    def zbody(j, carry):
        hist_v[pl.ds(j * 16, 16)] = zero16
        return carry

    lax.fori_loop(0, NPAD // 16, zbody, 0)
    cp.wait()
    ones16 = jnp.ones((16,), jnp.float32)

    def inner(j, c2):
        idx = chunk_v[pl.ds(j * 16, 16)]
        plsc.addupdate_scatter(hist_v, [idx], ones16)
        return c2

    lax.fori_loop(0, EPT // 16, inner, 0)
    pltpu.sync_copy(hist_v, out_hbm.at[wid])


@functools.partial(
    pl.kernel,
    mesh=_mesh,
    out_type=jax.ShapeDtypeStruct((NC, NACC, H), jnp.float32),
    scratch_types=[
        pltpu.VMEM((CPW, EC), jnp.int32),
        pltpu.VMEM((CPW, EC), jnp.int32),
        [pltpu.VMEM((EC, H), jnp.float32)] * NB,
        pltpu.VMEM_SHARED((NACC, H), jnp.float32),
        pltpu.SemaphoreType.DMA,
        [pltpu.SemaphoreType.DMA] * NB,
    ],
    compiler_params=_sc_params,
)
def _sc_agg(hp_hbm, src_hbm, dst_hbm, out_hbm,
            srcb, dstb, rows, accum_sh, isem, gsems):
    c = lax.axis_index("c")
    s = lax.axis_index("s")
    wid = s * NC + c

    icp1 = pltpu.async_copy(src_hbm.at[pl.ds(wid * CPW, CPW)], srcb, isem)
    icp2 = pltpu.async_copy(dst_hbm.at[pl.ds(wid * CPW, CPW)], dstb, isem)

    zero16 = jnp.zeros((16,), jnp.float32)

    def zbody(j, carry):
        rows[0][j // (H // 16), pl.ds((j % (H // 16)) * 16, 16)] = zero16
        return carry

    lax.fori_loop(0, EC * H // 16, zbody, 0)
    for q in range(RPT // EC):
        pltpu.sync_copy(rows[0], accum_sh.at[pl.ds(s * RPT + q * EC, EC)])
    icp1.wait()
    icp2.wait()
    plsc.subcore_barrier()

    for b in range(NB):
        pltpu.async_copy(hp_hbm.at[srcb.at[b]], rows[b], gsems[b])

    def body(i, carry):
        j0 = i * NB
        for b in range(NB):
            j = j0 + b
            pltpu.make_async_copy(hp_hbm.at[srcb.at[j]], rows[b],
                                  gsems[b]).wait()
            pltpu.sync_copy(rows[b], accum_sh.at[dstb.at[j]], add=True)

            @pl.when(j + NB < CPW)
            def _():
                pltpu.async_copy(hp_hbm.at[srcb.at[j + NB]], rows[b],
                                 gsems[b])
        return carry

    lax.fori_loop(0, CPW // NB, body, 0)
    plsc.subcore_barrier()
    pltpu.sync_copy(accum_sh.at[pl.ds(s * RPT, RPT)],
                    out_hbm.at[c, pl.ds(s * RPT, RPT)])


def _pair_expand(dvp):
    r2 = lax.broadcasted_iota(jnp.int32, (2, 128), 0)
    ll = lax.broadcasted_iota(jnp.int32, (2, 128), 1)
    sel = (ll // 64 == r2).astype(jnp.float32)
    return lax.dot_general(dvp, sel, (((1,), (0,)), ((), ())),
                           preferred_element_type=jnp.float32)


def _tc_dinv_body(degp_ref, dinv_ref):
    v = jnp.reshape(degp_ref[...], (NW, NPAD // 128, 128))
    deg = jnp.sum(v, axis=0) + 1.0
    dinv_ref[...] = lax.rsqrt(deg)


def _tc_first_body(xv_ref, w_ref, dvp_ref, hp_ref):
    dinvw = _pair_expand(dvp_ref[...])
    h = jnp.dot(xv_ref[...], w_ref[...], preferred_element_type=jnp.float32)
    hp_ref[...] = h * dinvw


def _tc_mid_body(aggp_ref, hp_ref, dvp_ref, b_ref, w_ref, out_ref):
    dinvw = _pair_expand(dvp_ref[...])
    a = aggp_ref[...]
    agg = a[0] + a[1] + hp_ref[...]
    z = jnp.maximum(agg * dinvw + b_ref[...], 0.0)
    out_ref[...] = jnp.dot(
        z, w_ref[...], preferred_element_type=jnp.float32) * dinvw


def _tc_pool_body(aggp_ref, hp_ref, dvp_ref, b_ref, batch_ref,
                  linw_ref, linb_ref, out_ref, acc_ref):
    i = pl.program_id(0)
    dinvw = _pair_expand(dvp_ref[...])
    a = aggp_ref[...]
    z = (a[0] + a[1] + hp_ref[...]) * dinvw + b_ref[...]
    bb = batch_ref[...]
    gi = lax.broadcasted_iota(jnp.int32, (_BW, G), 1)
    oh_e = (gi == bb[:, :1]).astype(jnp.float32)
    oh_o = (gi == bb[:, 1:]).astype(jnp.float32)
    ones = jnp.ones((_BW, 1), jnp.float32)
    zc_e = jnp.concatenate([z[:, :H], ones], axis=1)
    zc_o = jnp.concatenate([z[:, H:], ones], axis=1)
    part = (lax.dot_general(oh_e, zc_e, (((0,), (0,)), ((), ())),
                            preferred_element_type=jnp.float32)
            + lax.dot_general(oh_o, zc_o, (((0,), (0,)), ((), ())),
                              preferred_element_type=jnp.float32))

    @pl.when(i == 0)
    def _():
        acc_ref[...] = part

    @pl.when(i > 0)
    def _():
        acc_ref[...] = acc_ref[...] + part

    @pl.when(i == pl.num_programs(0) - 1)
    def _():
        sums = acc_ref[:, :H]
        cnt = acc_ref[:, H:]
        pooled = sums / jnp.maximum(cnt, 1.0)
        out_ref[...] = jnp.dot(
            pooled, linw_ref[...],
            preferred_element_type=jnp.float32) + linb_ref[...]


def _tc_dinv(degpv):
    return pl.pallas_call(
        _tc_dinv_body,
        grid=(1,),
        in_specs=[pl.BlockSpec((NW * NPAD // 128, 128), lambda i: (0, 0))],
        out_specs=pl.BlockSpec((NPAD // 128, 128), lambda i: (0, 0)),
        out_shape=jax.ShapeDtypeStruct((NPAD // 128, 128), jnp.float32),
    )(degpv)


def _tc_first(xv, wblk, dvp):
    return pl.pallas_call(
        _tc_first_body,
        grid=(N // BR,),
        in_specs=[
            pl.BlockSpec((_BW, 2 * F_IN), lambda i: (i, 0)),
            pl.BlockSpec((2 * F_IN, 128), lambda i: (0, 0)),
            pl.BlockSpec((_BW, 2), lambda i: (i, 0)),
        ],
        out_specs=pl.BlockSpec((_BW, 128), lambda i: (i, 0)),
        out_shape=jax.ShapeDtypeStruct((_VR, 128), jnp.float32),
    )(xv, wblk, dvp)


def _tc_mid(aggv, hpw, dvp, bb, wblk):
    return pl.pallas_call(
        _tc_mid_body,
        grid=(N // BR,),
        in_specs=[
            pl.BlockSpec((NC, _BW, 128), lambda i: (0, i, 0)),
            pl.BlockSpec((_BW, 128), lambda i: (i, 0)),
            pl.BlockSpec((_BW, 2), lambda i: (i, 0)),
            pl.BlockSpec((1, 128), lambda i: (0, 0)),
            pl.BlockSpec((128, 128), lambda i: (0, 0)),
        ],
        out_specs=pl.BlockSpec((_BW, 128), lambda i: (i, 0)),
        out_shape=jax.ShapeDtypeStruct((_VR, 128), jnp.float32),
    )(aggv, hpw, dvp, bb, wblk)


def _tc_pool(aggv, hpw, dvp, bb, batch2, linw, linb):
    return pl.pallas_call(
        _tc_pool_body,
        grid=(N // BR,),
        in_specs=[
            pl.BlockSpec((NC, _BW, 128), lambda i: (0, i, 0)),
            pl.BlockSpec((_BW, 128), lambda i: (i, 0)),
            pl.BlockSpec((_BW, 2), lambda i: (i, 0)),
            pl.BlockSpec((1, 128), lambda i: (0, 0)),
            pl.BlockSpec((_BW, 2), lambda i: (i, 0)),
            pl.BlockSpec((H, C), lambda i: (0, 0)),
            pl.BlockSpec((1, C), lambda i: (0, 0)),
        ],
        out_specs=pl.BlockSpec((G, C), lambda i: (0, 0)),
        out_shape=jax.ShapeDtypeStruct((G, C), jnp.float32),
        scratch_shapes=[pltpu.VMEM((G, H + 1), jnp.float32)],
    )(aggv, hpw, dvp, bb, batch2, linw, linb)


def _blockdiag(w):
    k, m = w.shape
    out = jnp.zeros((2 * k, 2 * m), jnp.float32)
    return out.at[:k, :m].set(w).at[k:, m:].set(w)


def kernel(x, edge_index, batch, W1, b1, W2, b2, W3, b3, lin_W, lin_b):
    src2 = edge_index[0].reshape(E // EC, EC)
    dst = edge_index[1]
    dst2 = dst.reshape(E // EC, EC)
    degpv = _sc_hist(dst).reshape(NW * NPAD // 128, 128)
    dvp = _tc_dinv(degpv).reshape(NPAD)[:N].reshape(N // 2, 2)
    xv = x.reshape(N // 2, 2 * F_IN)
    bb1 = jnp.concatenate([b1, b1]).reshape(1, 128)
    bb2 = jnp.concatenate([b2, b2]).reshape(1, 128)
    bb3 = jnp.concatenate([b3, b3]).reshape(1, 128)
    hpw1 = _tc_first(xv, _blockdiag(W1), dvp)
    agg1 = _sc_agg(hpw1.reshape(N, H), src2, dst2)
    hpw2 = _tc_mid(agg1.reshape(NC, NACC * H // 128, 128), hpw1, dvp,
                   bb1, _blockdiag(W2))
    agg2 = _sc_agg(hpw2.reshape(N, H), src2, dst2)
    hpw3 = _tc_mid(agg2.reshape(NC, NACC * H // 128, 128), hpw2, dvp,
                   bb2, _blockdiag(W3))
    agg3 = _sc_agg(hpw3.reshape(N, H), src2, dst2)
    return _tc_pool(agg3.reshape(NC, NACC * H // 128, 128), hpw3, dvp,
                    bb3, batch.reshape(N // 2, 2),
                    lin_W, lin_b.reshape(1, C))
